# baseline (speedup 1.0000x reference)
import functools

import jax
import jax.numpy as jnp
from jax import lax
from jax.experimental import pallas as pl
from jax.experimental.pallas import tpu as pltpu
from jax.experimental.pallas import tpu_sc as plsc

_B = 4096
_K = 8192
_D = 64
_BM = 512
_DP = 128


def _argmin_body(x_ref, lm_ref, a2_ref, b2_ref, idx_ref, tp_ref, d2_ref):
    _argmin_common(x_ref, lm_ref, a2_ref, b2_ref, idx_ref, d2_ref)
    kb = tp_ref.shape[0]
    rows = lm_ref[pl.ds(pl.program_id(0) * kb, kb), :]
    tp_ref[...] = jnp.concatenate(
        [rows, jnp.zeros((kb, _DP - _D), jnp.float32)], axis=1)


def _argmin_body_notable(x_ref, lm_ref, a2_ref, b2_ref, idx_ref, d2_ref):
    _argmin_common(x_ref, lm_ref, a2_ref, b2_ref, idx_ref, d2_ref)


def _argmin_common(x_ref, lm_ref, a2_ref, b2_ref, idx_ref, d2_ref):
    xm2 = x_ref[...] * -2.0
    a2 = a2_ref[...]
    s = lax.dot_general(xm2, lm_ref[...], (((1,), (1,)), ((), ())),
                        preferred_element_type=jnp.float32)
    nf = _K // 128
    rc = 64
    rm2_parts = []
    for c in range(s.shape[0] // rc):
        rows = slice(c * rc, (c + 1) * rc)
        a2c = a2[rows]
        m = None
        for f in range(nf):
            cols = slice(f * 128, (f + 1) * 128)
            blk = (a2c + b2_ref[:, cols]) + s[rows, cols]
            d2_ref[rows, cols] = blk
            m = blk if m is None else jnp.minimum(m, blk)
        rm2_parts.append(jnp.min(m, axis=1, keepdims=True))
    rm2 = jnp.concatenate(rm2_parts, axis=0)
    d2 = d2_ref[...]
    r = jnp.sqrt(jnp.maximum(rm2, 0.0))
    ib = lax.bitcast_convert_type(r * r, jnp.int32)
    karr = lax.broadcasted_iota(jnp.int32, (1, 14), 1) - 6
    cks = lax.bitcast_convert_type(ib + karr, jnp.float32)
    oks = jnp.sqrt(jnp.maximum(cks, 0.0)) <= r
    t = jnp.max(jnp.where(oks, cks, rm2), axis=1, keepdims=True)
    t = jnp.where(rm2 <= 0.0, 0.0, t)
    nf = _K // 128
    rc = 64
    lane = lax.broadcasted_iota(jnp.int32, (rc, 128), 1)
    for c in range(d2.shape[0] // rc):
        rows = slice(c * rc, (c + 1) * rc)
        tc_ = t[rows]
        fm = jnp.full((rc, 128), nf, jnp.int32)
        for f in range(nf - 1, -1, -1):
            fm = jnp.where(d2[rows, f * 128:(f + 1) * 128] <= tc_, f, fm)
        idx_ref[pl.ds(c * rc, rc)] = jnp.min(fm * 128 + lane, axis=1)


_IN_SPECS = [
    pl.BlockSpec((_BM, _D), lambda i: (i, 0)),
    pl.BlockSpec((_K, _D), lambda i: (0, 0)),
    pl.BlockSpec((_BM, 1), lambda i: (i, 0)),
    pl.BlockSpec((1, _K), lambda i: (0, 0)),
]


def _argmin_tc(x, landmarks, a2, b2, nb):
    return pl.pallas_call(
        _argmin_body,
        grid=(nb // _BM,),
        in_specs=_IN_SPECS,
        out_specs=[
            pl.BlockSpec((_BM,), lambda i: (i,)),
            pl.BlockSpec((_K * _BM // nb, _DP), lambda i: (i, 0)),
        ],
        out_shape=[
            jax.ShapeDtypeStruct((nb,), jnp.int32),
            jax.ShapeDtypeStruct((_K, _DP), jnp.float32),
        ],
        scratch_shapes=[pltpu.VMEM((_BM, _K), jnp.float32)],
        compiler_params=pltpu.CompilerParams(
            dimension_semantics=("parallel",)),
    )(x, landmarks, a2, b2)


def _argmin_tc_notable(x, landmarks, a2, b2, nb):
    return pl.pallas_call(
        _argmin_body_notable,
        grid=(nb // _BM,),
        in_specs=_IN_SPECS,
        out_specs=pl.BlockSpec((_BM,), lambda i: (i,)),
        out_shape=jax.ShapeDtypeStruct((nb,), jnp.int32),
        scratch_shapes=[pltpu.VMEM((_BM, _K), jnp.float32)],
        compiler_params=pltpu.CompilerParams(
            dimension_semantics=("parallel",)),
    )(x, landmarks, a2, b2)


def _gather_sc(table_pad, idx):
    info = plsc.get_sparse_core_info()
    nw = info.num_cores * info.num_subcores
    nb = idx.shape[0]
    b_per_w = nb // nw
    mesh = plsc.VectorSubcoreMesh(core_axis_name="c", subcore_axis_name="s")

    @functools.partial(
        pl.kernel, mesh=mesh,
        out_type=jax.ShapeDtypeStruct((nb, _DP), jnp.float32),
        scratch_types=[
            pltpu.VMEM((b_per_w,), jnp.int32),
            pltpu.VMEM((b_per_w, _DP), jnp.float32),
            pltpu.SemaphoreType.DMA,
        ],
    )
    def k(table_hbm, idx_hbm, out_hbm, idx_v, rows_v, sem):
        wid = lax.axis_index("s") * info.num_cores + lax.axis_index("c")
        base = wid * b_per_w
        pltpu.sync_copy(idx_hbm.at[pl.ds(base, b_per_w)], idx_v)
        pltpu.async_copy(table_hbm.at[idx_v], rows_v, sem).wait()
        pltpu.sync_copy(rows_v, out_hbm.at[pl.ds(base, b_per_w)])

    return k(table_pad, idx)


def kernel(x, landmarks):
    a2 = jnp.sum(x * x, axis=-1, keepdims=True)
    b2 = jnp.sum(landmarks * landmarks, axis=-1)[None, :]
    h = _B // 2
    idx0, table_pad = _argmin_tc(x[:h], landmarks, a2[:h], b2, h)
    out0 = _gather_sc(table_pad, idx0)
    idx1 = _argmin_tc_notable(x[h:], landmarks, a2[h:], b2, h)
    out1 = _gather_sc(table_pad, idx1)
    return jnp.concatenate([out0, out1], axis=0)[:, :_D]


# speedup vs baseline: 1.0546x; 1.0546x over previous
import functools

import jax
import jax.numpy as jnp
from jax import lax
from jax.experimental import pallas as pl
from jax.experimental.pallas import tpu as pltpu
from jax.experimental.pallas import tpu_sc as plsc

_B = 4096
_K = 8192
_D = 64
_BM = 512
_DP = 128


def _argmin_body(x_ref, lm_ref, a2_ref, b2_ref, idx_ref, tp_ref, d2_ref):
    _argmin_common(x_ref, lm_ref, a2_ref, b2_ref, idx_ref, d2_ref)
    kb = tp_ref.shape[0]
    rows = lm_ref[pl.ds(pl.program_id(0) * kb, kb), :]
    tp_ref[...] = jnp.concatenate(
        [rows, jnp.zeros((kb, _DP - _D), jnp.float32)], axis=1)


def _argmin_common(x_ref, lm_ref, a2_ref, b2_ref, idx_ref, d2_ref):
    xm2 = x_ref[...] * -2.0
    a2 = a2_ref[...]
    s = lax.dot_general(xm2, lm_ref[...], (((1,), (1,)), ((), ())),
                        preferred_element_type=jnp.float32)
    nf = _K // 128
    rc = 64
    rm2_parts = []
    for c in range(s.shape[0] // rc):
        rows = slice(c * rc, (c + 1) * rc)
        a2c = a2[rows]
        m = None
        for f in range(nf):
            cols = slice(f * 128, (f + 1) * 128)
            blk = (a2c + b2_ref[:, cols]) + s[rows, cols]
            d2_ref[rows, cols] = blk
            m = blk if m is None else jnp.minimum(m, blk)
        rm2_parts.append(jnp.min(m, axis=1, keepdims=True))
    rm2 = jnp.concatenate(rm2_parts, axis=0)
    d2 = d2_ref[...]
    r = jnp.sqrt(jnp.maximum(rm2, 0.0))
    ib = lax.bitcast_convert_type(r * r, jnp.int32)
    karr = lax.broadcasted_iota(jnp.int32, (1, 14), 1) - 6
    cks = lax.bitcast_convert_type(ib + karr, jnp.float32)
    oks = jnp.sqrt(jnp.maximum(cks, 0.0)) <= r
    t = jnp.max(jnp.where(oks, cks, rm2), axis=1, keepdims=True)
    t = jnp.where(rm2 <= 0.0, 0.0, t)
    nf = _K // 128
    rc = 64
    lane = lax.broadcasted_iota(jnp.int32, (rc, 128), 1)
    for c in range(d2.shape[0] // rc):
        rows = slice(c * rc, (c + 1) * rc)
        tc_ = t[rows]
        fm = jnp.full((rc, 128), nf, jnp.int32)
        for f in range(nf - 1, -1, -1):
            fm = jnp.where(d2[rows, f * 128:(f + 1) * 128] <= tc_, f, fm)
        idx_ref[pl.ds(c * rc, rc)] = jnp.min(fm * 128 + lane, axis=1)


_IN_SPECS = [
    pl.BlockSpec((_BM, _D), lambda i: (i, 0)),
    pl.BlockSpec((_K, _D), lambda i: (0, 0)),
    pl.BlockSpec((_BM, 1), lambda i: (i, 0)),
    pl.BlockSpec((1, _K), lambda i: (0, 0)),
]


def _argmin_tc(x, landmarks, a2, b2, nb):
    return pl.pallas_call(
        _argmin_body,
        grid=(nb // _BM,),
        in_specs=_IN_SPECS,
        out_specs=[
            pl.BlockSpec((_BM,), lambda i: (i,)),
            pl.BlockSpec((_K * _BM // nb, _DP), lambda i: (i, 0)),
        ],
        out_shape=[
            jax.ShapeDtypeStruct((nb,), jnp.int32),
            jax.ShapeDtypeStruct((_K, _DP), jnp.float32),
        ],
        scratch_shapes=[pltpu.VMEM((_BM, _K), jnp.float32)],
        compiler_params=pltpu.CompilerParams(
            dimension_semantics=("parallel",)),
    )(x, landmarks, a2, b2)


def _gather_sc(table_pad, idx):
    info = plsc.get_sparse_core_info()
    nw = info.num_cores * info.num_subcores
    nb = idx.shape[0]
    b_per_w = nb // nw
    mesh = plsc.VectorSubcoreMesh(core_axis_name="c", subcore_axis_name="s")

    @functools.partial(
        pl.kernel, mesh=mesh,
        out_type=jax.ShapeDtypeStruct((nb, _DP), jnp.float32),
        scratch_types=[
            pltpu.VMEM((b_per_w,), jnp.int32),
            pltpu.VMEM((b_per_w, _DP), jnp.float32),
            pltpu.SemaphoreType.DMA,
        ],
    )
    def k(table_hbm, idx_hbm, out_hbm, idx_v, rows_v, sem):
        wid = lax.axis_index("s") * info.num_cores + lax.axis_index("c")
        base = wid * b_per_w
        pltpu.sync_copy(idx_hbm.at[pl.ds(base, b_per_w)], idx_v)
        pltpu.async_copy(table_hbm.at[idx_v], rows_v, sem).wait()
        pltpu.sync_copy(rows_v, out_hbm.at[pl.ds(base, b_per_w)])

    return k(table_pad, idx)


def kernel(x, landmarks):
    a2 = jnp.sum(x * x, axis=-1, keepdims=True)
    b2 = jnp.sum(landmarks * landmarks, axis=-1)[None, :]
    idx, table_pad = _argmin_tc(x, landmarks, a2, b2, _B)
    return _gather_sc(table_pad, idx)[:, :_D]
